# Initial kernel scaffold; baseline (speedup 1.0000x reference)
import jax
import jax.numpy as jnp
from jax.experimental import pallas as pl


def kernel(TensorA, TensorB):
    raise NotImplementedError("write your pallas kernel here")


# speedup vs baseline: 4.2423x; 4.2423x over previous
import jax
import jax.numpy as jnp
from jax.experimental import pallas as pl
from jax.experimental.pallas import tpu as pltpu

_DIAG_W = 0.707


def _shl_e(t):
    return jnp.concatenate([t[:, :1], t[:, :-1]], axis=1)


def _shr_e(t):
    return jnp.concatenate([t[:, 1:], t[:, -1:]], axis=1)


def _shl_z(t):
    z = jnp.zeros((t.shape[0], 1), t.dtype)
    return jnp.concatenate([z, t[:, :-1]], axis=1)


def _shr_z(t):
    z = jnp.zeros((t.shape[0], 1), t.dtype)
    return jnp.concatenate([t[:, 1:], z], axis=1)


def _nsml_strip(ref, r0, r1, h, w):
    a = max(r0 - 1, 0)
    b = min(r1 + 1, h)
    lo = a - 1
    hi = b + 1
    parts = []
    if lo < 0:
        parts.append(ref[0, 0:1, :])
    parts.append(ref[0, max(lo, 0):min(hi, h), :])
    if hi > h:
        parts.append(ref[0, h - 1:h, :])
    x = jnp.concatenate(parts, axis=0) if len(parts) > 1 else parts[0]

    c = x[1:-1]
    up = x[:-2]
    dn = x[2:]
    ortho = (jnp.abs(c - up) + jnp.abs(c - dn)
             + jnp.abs(c - _shl_e(c)) + jnp.abs(c - _shr_e(c)))
    diag = (jnp.abs(c - _shl_e(up)) + jnp.abs(c - _shr_e(dn))
            + jnp.abs(c - _shr_e(up)) + jnp.abs(c - _shl_e(dn)))
    s = (ortho + _DIAG_W * diag) * 0.5
    sq = s * s

    zrow = jnp.zeros((1, w), jnp.float32)
    ps = []
    if r0 == 0:
        ps.append(zrow)
    ps.append(sq)
    if r1 == h:
        ps.append(zrow)
    sqe = jnp.concatenate(ps, axis=0) if len(ps) > 1 else ps[0]

    m = r1 - r0
    top = sqe[0:m]
    mid = sqe[1:m + 1]
    bot = sqe[2:m + 2]
    v = top + 2.0 * mid + bot
    return (_shl_z(v) + 2.0 * v + _shr_z(v)) * (1.0 / 16.0)


def _clar_loss(a3, b3, *, interpret=False):
    n, h, w = a3.shape
    strip = h // 4 if h % 4 == 0 else h
    inv_count = 1.0 / float(n * h * w)

    def body(a_ref, b_ref, o_ref, acc_ref):
        i = pl.program_id(0)

        @pl.when(i == 0)
        def _():
            acc_ref[...] = jnp.zeros_like(acc_ref)

        part = jnp.zeros((1, w), jnp.float32)
        for r0 in range(0, h, strip):
            na = _nsml_strip(a_ref, r0, r0 + strip, h, w)
            nb = _nsml_strip(b_ref, r0, r0 + strip, h, w)
            d = na - nb
            part = part + jnp.sum(d * d, axis=0, keepdims=True)
        acc_ref[...] += part

        @pl.when(i == n - 1)
        def _():
            total = jnp.sum(acc_ref[...]) * inv_count
            o_ref[...] = jnp.full((1, 1, w), total, jnp.float32)

    out = pl.pallas_call(
        body,
        grid=(n,),
        in_specs=[
            pl.BlockSpec((1, h, w), lambda i: (i, 0, 0)),
            pl.BlockSpec((1, h, w), lambda i: (i, 0, 0)),
        ],
        out_specs=pl.BlockSpec((1, 1, w), lambda i: (0, 0, 0)),
        out_shape=jax.ShapeDtypeStruct((1, 1, w), jnp.float32),
        scratch_shapes=[pltpu.VMEM((1, w), jnp.float32)],
        compiler_params=pltpu.CompilerParams(
            dimension_semantics=("arbitrary",),
            vmem_limit_bytes=48 * 1024 * 1024,
        ),
        name="clar_loss",
        interpret=interpret,
    )(a3, b3)
    return out[0, 0, 0]


def kernel(TensorA, TensorB):
    bsz, c, h, w = TensorA.shape
    a3 = TensorA.reshape(bsz * c, h, w)
    b3 = TensorB.reshape(bsz * c, h, w)
    return _clar_loss(a3, b3)
